# Initial kernel scaffold; baseline (speedup 1.0000x reference)
import jax
import jax.numpy as jnp
from jax.experimental import pallas as pl


def kernel(x, perm):
    raise NotImplementedError("write your pallas kernel here")


# speedup vs baseline: 6.8729x; 6.8729x over previous
import functools

import jax
import jax.numpy as jnp
from jax import lax
from jax.experimental import pallas as pl
from jax.experimental.pallas import tpu as pltpu
from jax.experimental.pallas import tpu_sc as plsc

B = 4
N = 2048 * 1024
NQ = 4
SLABW = N // NQ
NC, NS = 2, 16
JPW = N // (NC * NS)
SPW = SLABW // NS
JBLK = 4096
NJB = JPW // JBLK

_mesh = plsc.VectorSubcoreMesh(core_axis_name="c", subcore_axis_name="s")


@functools.partial(
    pl.kernel,
    out_type=(
        jax.ShapeDtypeStruct((NQ, B, N), jnp.float32),
        jax.ShapeDtypeStruct((B, N), jnp.float32),
    ),
    mesh=_mesh,
    scratch_types=[
        pltpu.VMEM((JPW,), jnp.int32),
        pltpu.VMEM((JBLK,), jnp.int32),
        pltpu.VMEM((JBLK,), jnp.float32),
        pltpu.VMEM_SHARED((SLABW,), jnp.float32),
        pltpu.SemaphoreType.DMA,
    ],
)
def _sc_gather(x_hbm, perm_hbm, part_hbm, y_hbm, loc_v, idx_v, vals_v,
               slab_sh, sem):
    c = lax.axis_index("c")
    s = lax.axis_index("s")
    wj = pl.multiple_of((s * NC + c) * JPW, JPW)

    for pb in range(JPW // JBLK):
        pltpu.sync_copy(perm_hbm.at[pl.ds(wj + pb * JBLK, JBLK)], idx_v)

        def _prep(i, carry):
            v = idx_v[pl.ds(i * 16, 16)]
            loc_v[pl.ds(pb * JBLK + i * 16, 16)] = lax.bitwise_and(
                v, SLABW - 1)
            return carry

        lax.fori_loop(0, JBLK // 16, _prep, 0)

    wjq = wj // SLABW
    wjl = pl.multiple_of(lax.bitwise_and(wj, SLABW - 1), JPW)

    for b in range(B):
        for q in range(NQ):
            sbase = pl.multiple_of(q * SLABW + s * SPW, 128)
            lbase = pl.multiple_of(s * SPW, 128)
            pltpu.sync_copy(x_hbm.at[b].at[pl.ds(sbase, SPW)],
                            slab_sh.at[pl.ds(lbase, SPW)])
            plsc.subcore_barrier()

            @pl.when(wjq == q)
            def _():
                pltpu.sync_copy(slab_sh.at[pl.ds(wjl, JPW)],
                                y_hbm.at[b].at[pl.ds(wj, JPW)])

            def _jblk(jb, carry):
                jo = pl.multiple_of(jb * JBLK, JBLK)
                pltpu.async_copy(
                    slab_sh.at[loc_v.at[pl.ds(jo, JBLK)]], vals_v,
                    sem).wait()
                pltpu.sync_copy(
                    vals_v, part_hbm.at[q].at[b].at[pl.ds(wj + jo, JBLK)])
                return carry

            lax.fori_loop(0, NJB, _jblk, 0)
            plsc.subcore_barrier()


_TCB = 512


def _tc_merge_body(perm_ref, part_ref, out_ref):
    q = (perm_ref[...] >> 19)[None]
    lo = jnp.where(q == 0, part_ref[0], part_ref[1])
    hi = jnp.where(q == 2, part_ref[2], part_ref[3])
    out_ref[...] = jnp.where(q < 2, lo, hi)


_tc_merge = pl.pallas_call(
    _tc_merge_body,
    grid=(N // 128 // _TCB,),
    in_specs=[
        pl.BlockSpec((_TCB, 128), lambda i: (i, 0)),
        pl.BlockSpec((NQ, B, _TCB, 128), lambda i: (0, 0, i, 0)),
    ],
    out_specs=pl.BlockSpec((B, _TCB, 128), lambda i: (0, i, 0)),
    out_shape=jax.ShapeDtypeStruct((B, N // 128, 128), jnp.float32),
)


def kernel(x, perm):
    xf = x.reshape(B, N)
    part, y = _sc_gather(xf, perm)
    xp = _tc_merge(
        perm.reshape(N // 128, 128),
        part.reshape(NQ, B, N // 128, 128),
    )
    return xp.reshape(x.shape), y.reshape(x.shape)


# speedup vs baseline: 7.3760x; 1.0732x over previous
import functools

import jax
import jax.numpy as jnp
from jax import lax
from jax.experimental import pallas as pl
from jax.experimental.pallas import tpu as pltpu
from jax.experimental.pallas import tpu_sc as plsc

B = 4
N = 2048 * 1024
NQ = 4
SLABW = N // NQ
NC, NS = 2, 16
JPW = N // (NC * NS)
SPW = SLABW // NS
JBLK = 4096
NJB = JPW // JBLK

_mesh = plsc.VectorSubcoreMesh(core_axis_name="c", subcore_axis_name="s")


@functools.partial(
    pl.kernel,
    out_type=(
        jax.ShapeDtypeStruct((NQ, B, N), jnp.float32),
        jax.ShapeDtypeStruct((B, N), jnp.float32),
    ),
    mesh=_mesh,
    scratch_types=[
        pltpu.VMEM((JPW,), jnp.int32),
        pltpu.VMEM((JBLK,), jnp.int32),
        pltpu.VMEM((JBLK,), jnp.float32),
        pltpu.VMEM((JBLK,), jnp.float32),
        pltpu.VMEM_SHARED((SLABW,), jnp.float32),
        pltpu.SemaphoreType.DMA,
    ],
)
def _sc_gather(x_hbm, perm_hbm, part_hbm, y_hbm, loc_v, idx_v, vals_a,
               vals_b, slab_sh, sem):
    vals_v = (vals_a, vals_b)
    c = lax.axis_index("c")
    s = lax.axis_index("s")
    wj = pl.multiple_of((s * NC + c) * JPW, JPW)

    for pb in range(JPW // JBLK):
        pltpu.sync_copy(perm_hbm.at[pl.ds(wj + pb * JBLK, JBLK)], idx_v)

        def _prep(i, carry):
            v = idx_v[pl.ds(i * 16, 16)]
            loc_v[pl.ds(pb * JBLK + i * 16, 16)] = lax.bitwise_and(
                v, SLABW - 1)
            return carry

        lax.fori_loop(0, JBLK // 16, _prep, 0)

    wjq = wj // SLABW
    wjl = pl.multiple_of(lax.bitwise_and(wj, SLABW - 1), JPW)

    for b in range(B):
        for q in range(NQ):
            sbase = pl.multiple_of(q * SLABW + s * SPW, 128)
            lbase = pl.multiple_of(s * SPW, 128)
            pltpu.sync_copy(x_hbm.at[b].at[pl.ds(sbase, SPW)],
                            slab_sh.at[pl.ds(lbase, SPW)])
            plsc.subcore_barrier()

            @pl.when(wjq == q)
            def _():
                pltpu.sync_copy(slab_sh.at[pl.ds(wjl, JPW)],
                                y_hbm.at[b].at[pl.ds(wj, JPW)])

            def _pair(jj, carry):
                for par in range(2):
                    jb = jj * 2 + par
                    jo = pl.multiple_of(jb * JBLK, JBLK)
                    pltpu.async_copy(
                        slab_sh.at[loc_v.at[pl.ds(jo, JBLK)]],
                        vals_v[par], sem)

                    @pl.when(jb > 0)
                    def _():
                        pjo = pl.multiple_of((jb - 1) * JBLK, JBLK)
                        pv = vals_v[1 - par]
                        pltpu.make_async_copy(
                            slab_sh.at[loc_v.at[pl.ds(pjo, JBLK)]], pv,
                            sem).wait()
                        pltpu.sync_copy(
                            pv,
                            part_hbm.at[q].at[b].at[pl.ds(wj + pjo, JBLK)])
                return carry

            lax.fori_loop(0, NJB // 2, _pair, 0)
            ljo = pl.multiple_of((NJB - 1) * JBLK, JBLK)
            lv = vals_v[(NJB - 1) % 2]
            pltpu.make_async_copy(
                slab_sh.at[loc_v.at[pl.ds(ljo, JBLK)]], lv, sem).wait()
            pltpu.sync_copy(
                lv, part_hbm.at[q].at[b].at[pl.ds(wj + ljo, JBLK)])
            plsc.subcore_barrier()


_TCB = 512


def _tc_merge_body(perm_ref, part_ref, out_ref):
    q = (perm_ref[...] >> 19)[None]
    lo = jnp.where(q == 0, part_ref[0], part_ref[1])
    hi = jnp.where(q == 2, part_ref[2], part_ref[3])
    out_ref[...] = jnp.where(q < 2, lo, hi)


_tc_merge = pl.pallas_call(
    _tc_merge_body,
    grid=(N // 128 // _TCB,),
    in_specs=[
        pl.BlockSpec((_TCB, 128), lambda i: (i, 0)),
        pl.BlockSpec((NQ, B, _TCB, 128), lambda i: (0, 0, i, 0)),
    ],
    out_specs=pl.BlockSpec((B, _TCB, 128), lambda i: (0, i, 0)),
    out_shape=jax.ShapeDtypeStruct((B, N // 128, 128), jnp.float32),
)


def kernel(x, perm):
    xf = x.reshape(B, N)
    part, y = _sc_gather(xf, perm)
    xp = _tc_merge(
        perm.reshape(N // 128, 128),
        part.reshape(NQ, B, N // 128, 128),
    )
    return xp.reshape(x.shape), y.reshape(x.shape)


# speedup vs baseline: 8.9329x; 1.2111x over previous
import functools

import jax
import jax.numpy as jnp
from jax import lax
from jax.experimental import pallas as pl
from jax.experimental.pallas import tpu as pltpu
from jax.experimental.pallas import tpu_sc as plsc

B = 4
N = 2048 * 1024
NH = 2
HALF = N // NH
NS = 16
JPW = N // NS
SPW = HALF // NS
JBLK = 8192
NJB = JPW // JBLK

_mesh = plsc.VectorSubcoreMesh(
    core_axis_name="c", subcore_axis_name="s", num_cores=1)


@functools.partial(
    pl.kernel,
    out_type=(
        jax.ShapeDtypeStruct((NH, B, N), jnp.float32),
        jax.ShapeDtypeStruct((B, N), jnp.float32),
        jax.ShapeDtypeStruct((N,), jnp.int32),
    ),
    mesh=_mesh,
    scratch_types=[
        pltpu.VMEM((JBLK,), jnp.int32),
        pltpu.VMEM((JBLK,), jnp.int32),
        pltpu.VMEM((JBLK,), jnp.int32),
        pltpu.VMEM((JBLK,), jnp.float32),
        pltpu.VMEM((JBLK,), jnp.float32),
        pltpu.VMEM_SHARED((HALF,), jnp.float32),
        pltpu.SemaphoreType.DMA,
    ],
)
def _sc_gather(x_hbm, perm_hbm, part_hbm, y_hbm, loc_hbm, idx_v, loc_a,
               loc_b, vals_a, vals_b, slab_sh, sem):
    loc_v = (loc_a, loc_b)
    vals_v = (vals_a, vals_b)
    s = lax.axis_index("s")
    wj = pl.multiple_of(s * JPW, JPW)

    for pb in range(NJB):
        po = pl.multiple_of(wj + pb * JBLK, JBLK)
        pltpu.sync_copy(perm_hbm.at[pl.ds(po, JBLK)], idx_v)

        def _prep(i, carry):
            v = idx_v[pl.ds(i * 16, 16)]
            idx_v[pl.ds(i * 16, 16)] = lax.bitwise_and(v, HALF - 1)
            return carry

        lax.fori_loop(0, JBLK // 16, _prep, 0)
        pltpu.sync_copy(idx_v, loc_hbm.at[pl.ds(po, JBLK)])

    wjh = wj // HALF
    wjl = pl.multiple_of(lax.bitwise_and(wj, HALF - 1), JPW)

    for b in range(B):
        for h in range(NH):
            sbase = pl.multiple_of(h * HALF + s * SPW, 128)
            lbase = pl.multiple_of(s * SPW, 128)
            pltpu.sync_copy(x_hbm.at[b].at[pl.ds(sbase, SPW)],
                            slab_sh.at[pl.ds(lbase, SPW)])
            plsc.subcore_barrier()

            @pl.when(wjh == h)
            def _():
                pltpu.sync_copy(slab_sh.at[pl.ds(wjl, JPW)],
                                y_hbm.at[b].at[pl.ds(wj, JPW)])

            def _pair(jj, carry):
                for par in range(2):
                    jb = jj * 2 + par
                    jo = pl.multiple_of(jb * JBLK, JBLK)
                    pltpu.sync_copy(loc_hbm.at[pl.ds(wj + jo, JBLK)],
                                    loc_v[par])
                    pltpu.async_copy(
                        slab_sh.at[loc_v[par]], vals_v[par], sem)

                    @pl.when(jb > 0)
                    def _():
                        pjo = pl.multiple_of((jb - 1) * JBLK, JBLK)
                        pv = vals_v[1 - par]
                        pltpu.make_async_copy(
                            slab_sh.at[loc_v[1 - par]], pv, sem).wait()
                        pltpu.sync_copy(
                            pv,
                            part_hbm.at[h].at[b].at[pl.ds(wj + pjo, JBLK)])
                return carry

            lax.fori_loop(0, NJB // 2, _pair, 0)
            ljo = pl.multiple_of((NJB - 1) * JBLK, JBLK)
            lv = vals_v[(NJB - 1) % 2]
            pltpu.make_async_copy(
                slab_sh.at[loc_v[(NJB - 1) % 2]], lv, sem).wait()
            pltpu.sync_copy(
                lv, part_hbm.at[h].at[b].at[pl.ds(wj + ljo, JBLK)])
            plsc.subcore_barrier()


_TCB = 512


def _tc_merge_body(perm_ref, part_ref, out_ref):
    hi = (perm_ref[...] >> 20)[None]
    out_ref[...] = jnp.where(hi == 0, part_ref[0], part_ref[1])


_tc_merge = pl.pallas_call(
    _tc_merge_body,
    grid=(N // 128 // _TCB,),
    in_specs=[
        pl.BlockSpec((_TCB, 128), lambda i: (i, 0)),
        pl.BlockSpec((NH, B, _TCB, 128), lambda i: (0, 0, i, 0)),
    ],
    out_specs=pl.BlockSpec((B, _TCB, 128), lambda i: (0, i, 0)),
    out_shape=jax.ShapeDtypeStruct((B, N // 128, 128), jnp.float32),
)


def kernel(x, perm):
    xf = x.reshape(B, N)
    part, y, _unused_loc = _sc_gather(xf, perm)
    xp = _tc_merge(
        perm.reshape(N // 128, 128),
        part.reshape(NH, B, N // 128, 128),
    )
    return xp.reshape(x.shape), y.reshape(x.shape)
